# baseline (speedup 1.0000x reference)
import functools

import jax
import jax.numpy as jnp
from jax import lax
from jax.experimental import pallas as pl
from jax.experimental.pallas import tpu as pltpu
from jax.experimental.pallas import tpu_sc as plsc

D = 64
NC, NS = 2, 16
NW = NC * NS
IW = 100
CH_IR = 4
CH = CH_IR * IW


def _pos_enc(seq_len, d_model):
    pos = jnp.arange(seq_len, dtype=jnp.float32)[:, None]
    i = jnp.arange(d_model // 2, dtype=jnp.float32)[None, :]
    div = jnp.exp(-(jnp.log(10000.0)) * (2.0 * i) / d_model)
    angles = pos * div
    pe = jnp.zeros((seq_len, d_model), dtype=jnp.float32)
    pe = pe.at[:, 0::2].set(jnp.sin(angles))
    pe = pe.at[:, 1::2].set(jnp.cos(angles))
    return pe


@functools.cache
def _make_kernel(n_rows_total):
    rows_per_w = n_rows_total // NW
    ir_per_w = rows_per_w // IW
    n_chunks = ir_per_w // CH_IR
    mesh = plsc.VectorSubcoreMesh(core_axis_name="c", subcore_axis_name="s",
                                  num_cores=NC, num_subcores=NS)

    @functools.partial(
        pl.kernel,
        out_type=jax.ShapeDtypeStruct((n_rows_total, D), jnp.float32),
        mesh=mesh,
        scratch_types=[
            pltpu.VMEM((CH_IR, IW), jnp.int32),
            pltpu.VMEM((CH, D), jnp.float32),
            pltpu.VMEM((CH, D), jnp.float32),
            pltpu.SemaphoreType.DMA,
        ],
        compiler_params=pltpu.CompilerParams(use_tc_tiling_on_sc=False),
    )
    def k(idx_hbm, table_hbm, pe_hbm, out_hbm, idx_v, rows_v, pe_v, sem):
        wid = lax.axis_index("s") * NC + lax.axis_index("c")
        ir_base = wid * ir_per_w
        row_base = wid * rows_per_w
        pltpu.sync_copy(pe_hbm, pe_v)

        @pl.loop(0, n_chunks)
        def _chunk(c):
            pltpu.sync_copy(idx_hbm.at[pl.ds(ir_base + c * CH_IR, CH_IR)],
                            idx_v)
            descs = [
                pltpu.async_copy(table_hbm.at[idx_v.at[j]],
                                 rows_v.at[pl.ds(j * IW, IW)], sem)
                for j in range(CH_IR)
            ]
            for dsc in descs:
                dsc.wait()

            @pl.loop(0, CH)
            def _row(r):
                for kk in range(D // 16):
                    sl = pl.ds(kk * 16, 16)
                    rows_v[r, sl] = rows_v[r, sl] + pe_v[r, sl]

            pltpu.sync_copy(rows_v,
                            out_hbm.at[pl.ds(row_base + c * CH, CH)])

    return k


def kernel(x, table):
    B, M = x.shape
    n = B * M
    idx = x.reshape(n // IW, IW).astype(jnp.int32)
    pe = _pos_enc(M, D)
    pe_t = jnp.tile(pe, (CH // M, 1))
    out = _make_kernel(n)(idx, table, pe_t)
    return out.reshape(B, M, D)


# speedup vs baseline: 1.0849x; 1.0849x over previous
import functools

import jax
import jax.numpy as jnp
from jax import lax
from jax.experimental import pallas as pl
from jax.experimental.pallas import tpu as pltpu
from jax.experimental.pallas import tpu_sc as plsc

D = 64
NC, NS = 2, 16
NW = NC * NS
IW = 100
CH_IR = 4
CH = CH_IR * IW


def _pos_enc(seq_len, d_model):
    pos = jnp.arange(seq_len, dtype=jnp.float32)[:, None]
    i = jnp.arange(d_model // 2, dtype=jnp.float32)[None, :]
    div = jnp.exp(-(jnp.log(10000.0)) * (2.0 * i) / d_model)
    angles = pos * div
    pe = jnp.zeros((seq_len, d_model), dtype=jnp.float32)
    pe = pe.at[:, 0::2].set(jnp.sin(angles))
    pe = pe.at[:, 1::2].set(jnp.cos(angles))
    return pe


@functools.cache
def _make_kernel(n_rows_total):
    rows_per_w = n_rows_total // NW
    ir_per_w = rows_per_w // IW
    n_chunks = ir_per_w // CH_IR
    mesh = plsc.VectorSubcoreMesh(core_axis_name="c", subcore_axis_name="s",
                                  num_cores=NC, num_subcores=NS)

    @functools.partial(
        pl.kernel,
        out_type=jax.ShapeDtypeStruct((n_rows_total, D), jnp.float32),
        mesh=mesh,
        scratch_types=[
            pltpu.VMEM((2, CH_IR, IW), jnp.int32),
            pltpu.VMEM((2, CH, D), jnp.float32),
            pltpu.VMEM((CH, D), jnp.float32),
            pltpu.SemaphoreType.DMA,
            pltpu.SemaphoreType.DMA,
            pltpu.SemaphoreType.DMA,
            pltpu.SemaphoreType.DMA,
        ],
        compiler_params=pltpu.CompilerParams(use_tc_tiling_on_sc=False),
    )
    def k(idx_hbm, table_hbm, pe_hbm, out_hbm, idx_v, rows_v, pe_v,
          gsem0, gsem1, osem0, osem1):
        gsem = (gsem0, gsem1)
        osem = (osem0, osem1)
        wid = lax.axis_index("s") * NC + lax.axis_index("c")
        ir_base = wid * ir_per_w
        row_base = wid * rows_per_w
        pltpu.sync_copy(pe_hbm, pe_v)

        def fire_gather(cc, b):
            pltpu.sync_copy(idx_hbm.at[pl.ds(ir_base + cc * CH_IR, CH_IR)],
                            idx_v.at[b])
            for j in range(CH_IR):
                pltpu.async_copy(table_hbm.at[idx_v.at[b, j]],
                                 rows_v.at[b, pl.ds(j * IW, IW)], gsem[b])

        def drain_gather(b):
            pltpu.make_async_copy(pe_hbm, rows_v.at[b], gsem[b]).wait()

        def drain_out(b):
            pltpu.make_async_copy(rows_v.at[b],
                                  out_hbm.at[pl.ds(row_base, CH)],
                                  osem[b]).wait()

        fire_gather(0, 0)

        @pl.loop(0, n_chunks, step=2)
        def _two(c):
            for b in (0, 1):
                cc = c + b
                ob = 1 - b
                drain_gather(b)

                @pl.when(cc + 1 < n_chunks)
                def _prefetch():
                    @pl.when(cc >= 1)
                    def _free_buf():
                        drain_out(ob)
                    fire_gather(cc + 1, ob)

                @plsc.parallel_loop(0, CH, unroll=8)
                def _row(r):
                    for kk in range(D // 16):
                        sl = pl.ds(kk * 16, 16)
                        rows_v[b, r, sl] = rows_v[b, r, sl] + pe_v[r, sl]

                pltpu.async_copy(rows_v.at[b],
                                 out_hbm.at[pl.ds(row_base + cc * CH, CH)],
                                 osem[b])

        drain_out(0)
        drain_out(1)

    return k


def kernel(x, table):
    B, M = x.shape
    n = B * M
    idx = x.reshape(n // IW, IW).astype(jnp.int32)
    pe = _pos_enc(M, D)
    pe_t = jnp.tile(pe, (CH // M, 1))
    out = _make_kernel(n)(idx, table, pe_t)
    return out.reshape(B, M, D)
